# baseline (speedup 1.0000x reference)
import functools

import jax
import jax.numpy as jnp
from jax import lax
from jax.experimental import pallas as pl
from jax.experimental.pallas import tpu as pltpu
from jax.experimental.pallas import tpu_sc as plsc

N_USERS = 50000
N_ITEMS = 50000
N_NODES = N_USERS + N_ITEMS
N_EDGES = 3200000
EMB = 16
N_LAYERS = 3

NC = 2
NS = 16
NW = NC * NS
CH = 512
NSTREAM = CH // 128
CPT = 198
NCHUNK_P = NW * CPT
N_PAD = NCHUNK_P * CH - N_EDGES
ROWS_A = 6248
ROWS_TAIL = N_NODES - NS * ROWS_A

_mesh = plsc.VectorSubcoreMesh(core_axis_name="c", subcore_axis_name="s")


@functools.partial(
    pl.kernel,
    out_type=jax.ShapeDtypeStruct((NC, N_NODES, EMB), jnp.float32),
    mesh=_mesh,
    scratch_types=[
        pltpu.VMEM((2, CH), jnp.int32),
        pltpu.VMEM((2, 2 * NSTREAM, 128), jnp.int32),
        pltpu.VMEM((CH, EMB), jnp.float32),
        pltpu.VMEM((CH, EMB), jnp.float32),
        pltpu.VMEM((CH, EMB), jnp.float32),
        pltpu.VMEM((3, NSTREAM, 128), jnp.int32),
        pltpu.VMEM_SHARED((N_NODES, EMB), jnp.float32),
        pltpu.SemaphoreType.DMA,
        pltpu.SemaphoreType.DMA,
        pltpu.SemaphoreType.DMA,
        pltpu.SemaphoreType.DMA,
        pltpu.SemaphoreType.DMA,
        pltpu.SemaphoreType.DMA,
        pltpu.SemaphoreType.DMA,
        pltpu.SemaphoreType.DMA,
    ],
    compiler_params=pltpu.CompilerParams(use_tc_tiling_on_sc=False,
                                         needs_layout_passes=False),
)
def _spmm(src_hbm, dw_hbm, ego_hbm, out_hbm,
          srcb, dwb, rows0, rows1, rows2, dsc, acc_sh,
          si0, si1, sg0, sg1, sg2, ss0, ss1, ss2):
    cid = lax.axis_index("c")
    sid = lax.axis_index("s")
    wid = cid * NS + sid
    base = wid * CPT

    rows = (rows0, rows1, rows2)
    sin = (si0, si1)
    sg = (sg0, sg1, sg2)
    ss = (ss0, ss1, ss2)

    def idx_start(b2, c):
        pltpu.async_copy(src_hbm.at[c], srcb.at[b2], sin[b2])
        pltpu.async_copy(dw_hbm.at[c], dwb.at[b2], sin[b2])

    def idx_wait(b2):
        pltpu.make_async_copy(src_hbm.at[0], srcb.at[b2], sin[b2]).wait()
        pltpu.make_async_copy(dw_hbm.at[0], dwb.at[b2], sin[b2]).wait()

    def gather_start(b3, b2):
        pltpu.async_copy(ego_hbm.at[srcb.at[b2]], rows[b3], sg[b3])

    def gather_wait(b3):
        pltpu.make_async_copy(ego_hbm.at[pl.ds(0, CH)], rows[b3],
                              sg[b3]).wait()

    def scale(b3, b2):
        rv = rows[b3]

        @pl.loop(0, CH // 16)
        def _scale(g):
            wv = plsc.bitcast(
                dwb[b2, NSTREAM + g // 8, pl.ds((g % 8) * 16, 16)],
                jnp.float32)
            e = g * 16
            for u in range(16):
                rv[e + u, :] = rv[e + u, :] * wv[u]

    def dst_copy(b3, b2):
        for j in range(NSTREAM):
            for g in range(8):
                dsc[b3, j, pl.ds(g * 16, 16)] = dwb[b2, j, pl.ds(g * 16, 16)]

    def scatter_start(b3):
        for j in range(NSTREAM):
            pltpu.async_copy(rows[b3].at[pl.ds(j * 128, 128)],
                             acc_sh.at[dsc.at[b3, j]], ss[b3], add=True)

    def scatter_wait(b3):
        pltpu.make_async_copy(rows[b3], out_hbm.at[0, pl.ds(0, CH)],
                              ss[b3]).wait()

    @pl.loop(0, CH)
    def _zero(i):
        rows0[i, :] = jnp.zeros((EMB,), jnp.float32)

    zbase = sid * ROWS_A
    nfull = ROWS_A // CH
    zrem = ROWS_A - nfull * CH
    for k in range(nfull):
        pltpu.sync_copy(rows0, acc_sh.at[pl.ds(zbase + k * CH, CH)])
    if zrem:
        pltpu.sync_copy(rows0.at[pl.ds(0, zrem)],
                        acc_sh.at[pl.ds(zbase + nfull * CH, zrem)])

    @pl.when(sid == NS - 1)
    def _zero_tail():
        pltpu.sync_copy(rows0.at[pl.ds(0, ROWS_TAIL)],
                        acc_sh.at[pl.ds(N_NODES - ROWS_TAIL, ROWS_TAIL)])

    plsc.subcore_barrier()

    idx_start(0, base)

    @pl.loop(0, CPT, step=6)
    def _rounds(j):
        for k in range(6):
            c = base + j + k
            b2 = k % 2
            b3 = k % 3
            pb2 = (k - 1) % 2
            pb3 = (k - 1) % 3

            idx_wait(b2)

            def _sw():
                scatter_wait(b3)

            if k < 3:
                pl.when(j > 0)(_sw)
            else:
                _sw()

            gather_start(b3, b2)

            def _drain_prev():
                gather_wait(pb3)
                scale(pb3, pb2)
                dst_copy(pb3, pb2)
                scatter_start(pb3)

            if k == 0:
                pl.when(j > 0)(_drain_prev)
            else:
                _drain_prev()

            if k == 5:
                @pl.when(j < CPT - 6)
                def _prefetch():
                    idx_start((k + 1) % 2, c + 1)
            else:
                idx_start((k + 1) % 2, c + 1)

    scatter_wait(0)
    gather_wait(2)
    scale(2, 1)
    dst_copy(2, 1)
    scatter_start(2)
    scatter_wait(1)
    scatter_wait(2)

    plsc.subcore_barrier()
    pltpu.sync_copy(acc_sh.at[pl.ds(zbase, ROWS_A)],
                    out_hbm.at[cid, pl.ds(zbase, ROWS_A)])

    @pl.when(sid == NS - 1)
    def _out_tail():
        pltpu.sync_copy(
            acc_sh.at[pl.ds(N_NODES - ROWS_TAIL, ROWS_TAIL)],
            out_hbm.at[cid, pl.ds(N_NODES - ROWS_TAIL, ROWS_TAIL)])


_R = N_NODES * EMB // 128
_BLK = 1024


def _combine_mid_body(p_ref, t_ref, ego_ref, tot_ref):
    s = p_ref[0] + p_ref[1]
    ego_ref[...] = s
    tot_ref[...] = t_ref[...] + s


def _combine_last_body(p_ref, t_ref, mean_ref):
    mean_ref[...] = (t_ref[...] + p_ref[0] + p_ref[1]) * (1.0 / (N_LAYERS + 1))


_grid = (pl.cdiv(_R, _BLK),)
_p_spec = pl.BlockSpec((NC, _BLK, 128), lambda i: (0, i, 0))
_m_spec = pl.BlockSpec((_BLK, 128), lambda i: (i, 0))

_combine_mid = pl.pallas_call(
    _combine_mid_body,
    grid=_grid,
    in_specs=[_p_spec, _m_spec],
    out_specs=[_m_spec, _m_spec],
    out_shape=[jax.ShapeDtypeStruct((_R, 128), jnp.float32)] * 2,
)

_combine_last = pl.pallas_call(
    _combine_last_body,
    grid=_grid,
    in_specs=[_p_spec, _m_spec],
    out_specs=_m_spec,
    out_shape=jax.ShapeDtypeStruct((_R, 128), jnp.float32),
)


def kernel(edge_index, edge_weight, user_emb, item_emb):
    ei = edge_index.astype(jnp.int32)
    pad = jnp.arange(N_PAD, dtype=jnp.int32) % N_NODES
    src = jnp.concatenate([ei[1], pad]).reshape(NCHUNK_P, CH)
    dst = jnp.concatenate([ei[0], pad]).reshape(NCHUNK_P, NSTREAM, 128)
    wbits = jax.lax.bitcast_convert_type(
        jnp.concatenate([edge_weight, jnp.zeros((N_PAD,), jnp.float32)]),
        jnp.int32).reshape(NCHUNK_P, NSTREAM, 128)
    dw = jnp.concatenate([dst, wbits], axis=1)

    ego = jnp.concatenate([user_emb, item_emb], axis=0)
    tot = ego.reshape(_R, 128)
    for layer in range(N_LAYERS):
        partials = _spmm(src, dw, ego)
        p = partials.reshape(NC, _R, 128)
        if layer < N_LAYERS - 1:
            ego2, tot = _combine_mid(p, tot)
            ego = ego2.reshape(N_NODES, EMB)
        else:
            mean = _combine_last(p, tot).reshape(N_NODES, EMB)
    return (mean[:N_USERS], mean[N_USERS:])


# speedup vs baseline: 1.0015x; 1.0015x over previous
import functools

import jax
import jax.numpy as jnp
from jax import lax
from jax.experimental import pallas as pl
from jax.experimental.pallas import tpu as pltpu
from jax.experimental.pallas import tpu_sc as plsc

N_USERS = 50000
N_ITEMS = 50000
N_NODES = N_USERS + N_ITEMS
N_EDGES = 3200000
EMB = 16
N_LAYERS = 3

NC = 2
NS = 16
NW = NC * NS
CH = 512
NSTREAM = CH // 128
CPT = 198
NCHUNK_P = NW * CPT
N_PAD = NCHUNK_P * CH - N_EDGES
ROWS_A = 6248
ROWS_TAIL = N_NODES - NS * ROWS_A

_mesh = plsc.VectorSubcoreMesh(core_axis_name="c", subcore_axis_name="s")


@functools.partial(
    pl.kernel,
    out_type=jax.ShapeDtypeStruct((NC, N_NODES, EMB), jnp.float32),
    mesh=_mesh,
    scratch_types=[
        pltpu.VMEM((2, CH), jnp.int32),
        pltpu.VMEM((2, 2 * NSTREAM, 128), jnp.int32),
        pltpu.VMEM((CH, EMB), jnp.float32),
        pltpu.VMEM((CH, EMB), jnp.float32),
        pltpu.VMEM((CH, EMB), jnp.float32),
        pltpu.VMEM((3, NSTREAM, 128), jnp.int32),
        pltpu.VMEM_SHARED((N_NODES, EMB), jnp.float32),
        pltpu.SemaphoreType.DMA,
        pltpu.SemaphoreType.DMA,
        pltpu.SemaphoreType.DMA,
        pltpu.SemaphoreType.DMA,
        pltpu.SemaphoreType.DMA,
        pltpu.SemaphoreType.DMA,
        pltpu.SemaphoreType.DMA,
        pltpu.SemaphoreType.DMA,
    ],
    compiler_params=pltpu.CompilerParams(use_tc_tiling_on_sc=False,
                                         needs_layout_passes=False),
)
def _spmm(src_hbm, dw_hbm, ego_hbm, out_hbm,
          srcb, dwb, rows0, rows1, rows2, dsc, acc_sh,
          si0, si1, sg0, sg1, sg2, ss0, ss1, ss2):
    cid = lax.axis_index("c")
    sid = lax.axis_index("s")
    wid = cid * NS + sid
    base = wid * CPT

    rows = (rows0, rows1, rows2)
    sin = (si0, si1)
    sg = (sg0, sg1, sg2)
    ss = (ss0, ss1, ss2)

    def idx_start(b2, c):
        pltpu.async_copy(src_hbm.at[c], srcb.at[b2], sin[b2])
        pltpu.async_copy(dw_hbm.at[c], dwb.at[b2], sin[b2])

    def idx_wait(b2):
        pltpu.make_async_copy(src_hbm.at[0], srcb.at[b2], sin[b2]).wait()
        pltpu.make_async_copy(dw_hbm.at[0], dwb.at[b2], sin[b2]).wait()

    def gather_start(b3, b2):
        for j in range(NSTREAM):
            pltpu.async_copy(ego_hbm.at[srcb.at[b2, pl.ds(j * 128, 128)]],
                             rows[b3].at[pl.ds(j * 128, 128)], sg[b3])

    def gather_wait(b3):
        pltpu.make_async_copy(ego_hbm.at[pl.ds(0, CH)], rows[b3],
                              sg[b3]).wait()

    def scale(b3, b2):
        rv = rows[b3]

        @pl.loop(0, CH // 16)
        def _scale(g):
            wv = plsc.bitcast(
                dwb[b2, NSTREAM + g // 8, pl.ds((g % 8) * 16, 16)],
                jnp.float32)
            e = g * 16
            for u in range(16):
                rv[e + u, :] = rv[e + u, :] * wv[u]

    def dst_copy(b3, b2):
        for j in range(NSTREAM):
            for g in range(8):
                dsc[b3, j, pl.ds(g * 16, 16)] = dwb[b2, j, pl.ds(g * 16, 16)]

    def scatter_start(b3):
        for j in range(NSTREAM):
            pltpu.async_copy(rows[b3].at[pl.ds(j * 128, 128)],
                             acc_sh.at[dsc.at[b3, j]], ss[b3], add=True)

    def scatter_wait(b3):
        pltpu.make_async_copy(rows[b3], out_hbm.at[0, pl.ds(0, CH)],
                              ss[b3]).wait()

    @pl.loop(0, CH)
    def _zero(i):
        rows0[i, :] = jnp.zeros((EMB,), jnp.float32)

    zbase = sid * ROWS_A
    nfull = ROWS_A // CH
    zrem = ROWS_A - nfull * CH
    for k in range(nfull):
        pltpu.sync_copy(rows0, acc_sh.at[pl.ds(zbase + k * CH, CH)])
    if zrem:
        pltpu.sync_copy(rows0.at[pl.ds(0, zrem)],
                        acc_sh.at[pl.ds(zbase + nfull * CH, zrem)])

    @pl.when(sid == NS - 1)
    def _zero_tail():
        pltpu.sync_copy(rows0.at[pl.ds(0, ROWS_TAIL)],
                        acc_sh.at[pl.ds(N_NODES - ROWS_TAIL, ROWS_TAIL)])

    plsc.subcore_barrier()

    idx_start(0, base)

    @pl.loop(0, CPT, step=6)
    def _rounds(j):
        for k in range(6):
            c = base + j + k
            b2 = k % 2
            b3 = k % 3
            pb2 = (k - 1) % 2
            pb3 = (k - 1) % 3

            idx_wait(b2)

            def _sw():
                scatter_wait(b3)

            if k < 3:
                pl.when(j > 0)(_sw)
            else:
                _sw()

            gather_start(b3, b2)

            def _drain_prev():
                gather_wait(pb3)
                scale(pb3, pb2)
                dst_copy(pb3, pb2)
                scatter_start(pb3)

            if k == 0:
                pl.when(j > 0)(_drain_prev)
            else:
                _drain_prev()

            if k == 5:
                @pl.when(j < CPT - 6)
                def _prefetch():
                    idx_start((k + 1) % 2, c + 1)
            else:
                idx_start((k + 1) % 2, c + 1)

    scatter_wait(0)
    gather_wait(2)
    scale(2, 1)
    dst_copy(2, 1)
    scatter_start(2)
    scatter_wait(1)
    scatter_wait(2)

    plsc.subcore_barrier()
    pltpu.sync_copy(acc_sh.at[pl.ds(zbase, ROWS_A)],
                    out_hbm.at[cid, pl.ds(zbase, ROWS_A)])

    @pl.when(sid == NS - 1)
    def _out_tail():
        pltpu.sync_copy(
            acc_sh.at[pl.ds(N_NODES - ROWS_TAIL, ROWS_TAIL)],
            out_hbm.at[cid, pl.ds(N_NODES - ROWS_TAIL, ROWS_TAIL)])


_R = N_NODES * EMB // 128
_BLK = 1024


def _combine_mid_body(p_ref, t_ref, ego_ref, tot_ref):
    s = p_ref[0] + p_ref[1]
    ego_ref[...] = s
    tot_ref[...] = t_ref[...] + s


def _combine_last_body(p_ref, t_ref, mean_ref):
    mean_ref[...] = (t_ref[...] + p_ref[0] + p_ref[1]) * (1.0 / (N_LAYERS + 1))


_grid = (pl.cdiv(_R, _BLK),)
_p_spec = pl.BlockSpec((NC, _BLK, 128), lambda i: (0, i, 0))
_m_spec = pl.BlockSpec((_BLK, 128), lambda i: (i, 0))

_combine_mid = pl.pallas_call(
    _combine_mid_body,
    grid=_grid,
    in_specs=[_p_spec, _m_spec],
    out_specs=[_m_spec, _m_spec],
    out_shape=[jax.ShapeDtypeStruct((_R, 128), jnp.float32)] * 2,
)

_combine_last = pl.pallas_call(
    _combine_last_body,
    grid=_grid,
    in_specs=[_p_spec, _m_spec],
    out_specs=_m_spec,
    out_shape=jax.ShapeDtypeStruct((_R, 128), jnp.float32),
)


def kernel(edge_index, edge_weight, user_emb, item_emb):
    ei = edge_index.astype(jnp.int32)
    pad = jnp.arange(N_PAD, dtype=jnp.int32) % N_NODES
    src = jnp.concatenate([ei[1], pad]).reshape(NCHUNK_P, CH)
    dst = jnp.concatenate([ei[0], pad]).reshape(NCHUNK_P, NSTREAM, 128)
    wbits = jax.lax.bitcast_convert_type(
        jnp.concatenate([edge_weight, jnp.zeros((N_PAD,), jnp.float32)]),
        jnp.int32).reshape(NCHUNK_P, NSTREAM, 128)
    dw = jnp.concatenate([dst, wbits], axis=1)

    ego = jnp.concatenate([user_emb, item_emb], axis=0)
    tot = ego.reshape(_R, 128)
    for layer in range(N_LAYERS):
        partials = _spmm(src, dw, ego)
        p = partials.reshape(NC, _R, 128)
        if layer < N_LAYERS - 1:
            ego2, tot = _combine_mid(p, tot)
            ego = ego2.reshape(N_NODES, EMB)
        else:
            mean = _combine_last(p, tot).reshape(N_NODES, EMB)
    return (mean[:N_USERS], mean[N_USERS:])


# speedup vs baseline: 1.0125x; 1.0109x over previous
import functools

import jax
import jax.numpy as jnp
from jax import lax
from jax.experimental import pallas as pl
from jax.experimental.pallas import tpu as pltpu
from jax.experimental.pallas import tpu_sc as plsc

N_USERS = 50000
N_ITEMS = 50000
N_NODES = N_USERS + N_ITEMS
N_EDGES = 3200000
EMB = 16
N_LAYERS = 3

NC = 2
NS = 16
NW = NC * NS
CH = 512
NSTREAM = CH // 128
CPT = 198
NCHUNK_P = NW * CPT
N_PAD = NCHUNK_P * CH - N_EDGES
ROWS_A = 6248
ROWS_TAIL = N_NODES - NS * ROWS_A

_mesh = plsc.VectorSubcoreMesh(core_axis_name="c", subcore_axis_name="s")


@functools.partial(
    pl.kernel,
    out_type=jax.ShapeDtypeStruct((NC, N_NODES, EMB), jnp.float32),
    mesh=_mesh,
    scratch_types=[
        pltpu.VMEM((2, NSTREAM, 128), jnp.int32),
        pltpu.VMEM((2, NSTREAM, 128), jnp.int32),
        pltpu.VMEM((2, CH // 16, 16), jnp.float32),
        pltpu.VMEM((CH, EMB), jnp.float32),
        pltpu.VMEM((CH, EMB), jnp.float32),
        pltpu.VMEM((CH, EMB), jnp.float32),
        pltpu.VMEM((3, NSTREAM, 128), jnp.int32),
        pltpu.VMEM_SHARED((N_NODES, EMB), jnp.float32),
        pltpu.SemaphoreType.DMA,
        pltpu.SemaphoreType.DMA,
        pltpu.SemaphoreType.DMA,
        pltpu.SemaphoreType.DMA,
        pltpu.SemaphoreType.DMA,
        pltpu.SemaphoreType.DMA,
        pltpu.SemaphoreType.DMA,
        pltpu.SemaphoreType.DMA,
    ],
    compiler_params=pltpu.CompilerParams(use_tc_tiling_on_sc=False),
)
def _spmm(src_hbm, dst_hbm, w_hbm, ego_hbm, out_hbm,
          srcb, dstb, wb, rows0, rows1, rows2, dsc, acc_sh,
          si0, si1, sg0, sg1, sg2, ss0, ss1, ss2):
    cid = lax.axis_index("c")
    sid = lax.axis_index("s")
    wid = cid * NS + sid
    base = wid * CPT

    rows = (rows0, rows1, rows2)
    sin = (si0, si1)
    sg = (sg0, sg1, sg2)
    ss = (ss0, ss1, ss2)

    def idx_start(b2, c):
        pltpu.async_copy(src_hbm.at[c], srcb.at[b2], sin[b2])
        pltpu.async_copy(dst_hbm.at[c], dstb.at[b2], sin[b2])
        pltpu.async_copy(w_hbm.at[c], wb.at[b2], sin[b2])

    def idx_wait(b2):
        pltpu.make_async_copy(src_hbm.at[0], srcb.at[b2], sin[b2]).wait()
        pltpu.make_async_copy(dst_hbm.at[0], dstb.at[b2], sin[b2]).wait()
        pltpu.make_async_copy(w_hbm.at[0], wb.at[b2], sin[b2]).wait()

    def gather_start(b3, b2):
        for j in range(NSTREAM):
            pltpu.async_copy(ego_hbm.at[srcb.at[b2, j]],
                             rows[b3].at[pl.ds(j * 128, 128)], sg[b3])

    def gather_wait(b3):
        pltpu.make_async_copy(ego_hbm.at[pl.ds(0, CH)], rows[b3],
                              sg[b3]).wait()

    def scale(b3, b2):
        rv = rows[b3]

        @pl.loop(0, CH // 16)
        def _scale(g):
            wv = wb[b2, g, :]
            e = g * 16
            for u in range(16):
                rv[e + u, :] = rv[e + u, :] * wv[u]

    def dst_copy(b3, b2):
        for j in range(NSTREAM):
            for g in range(8):
                dsc[b3, j, pl.ds(g * 16, 16)] = dstb[b2, j, pl.ds(g * 16, 16)]

    def scatter_start(b3):
        for j in range(NSTREAM):
            pltpu.async_copy(rows[b3].at[pl.ds(j * 128, 128)],
                             acc_sh.at[dsc.at[b3, j]], ss[b3], add=True)

    def scatter_wait(b3):
        for j in range(NSTREAM):
            pltpu.make_async_copy(rows[b3].at[pl.ds(j * 128, 128)],
                                  acc_sh.at[dsc.at[b3, j]], ss[b3]).wait()

    @pl.loop(0, CH)
    def _zero(i):
        rows0[i, :] = jnp.zeros((EMB,), jnp.float32)

    zbase = sid * ROWS_A
    nfull = ROWS_A // CH
    zrem = ROWS_A - nfull * CH
    for k in range(nfull):
        pltpu.sync_copy(rows0, acc_sh.at[pl.ds(zbase + k * CH, CH)])
    if zrem:
        pltpu.sync_copy(rows0.at[pl.ds(0, zrem)],
                        acc_sh.at[pl.ds(zbase + nfull * CH, zrem)])

    @pl.when(sid == NS - 1)
    def _zero_tail():
        pltpu.sync_copy(rows0.at[pl.ds(0, ROWS_TAIL)],
                        acc_sh.at[pl.ds(N_NODES - ROWS_TAIL, ROWS_TAIL)])

    plsc.subcore_barrier()

    idx_start(0, base)

    @pl.loop(0, CPT, step=6)
    def _rounds(j):
        for k in range(6):
            c = base + j + k
            b2 = k % 2
            b3 = k % 3
            pb2 = (k - 1) % 2
            pb3 = (k - 1) % 3

            idx_wait(b2)

            def _sw():
                scatter_wait(b3)

            if k < 3:
                pl.when(j > 0)(_sw)
            else:
                _sw()

            gather_start(b3, b2)

            def _drain_prev():
                gather_wait(pb3)
                scale(pb3, pb2)
                dst_copy(pb3, pb2)
                scatter_start(pb3)

            if k == 0:
                pl.when(j > 0)(_drain_prev)
            else:
                _drain_prev()

            if k == 5:
                @pl.when(j < CPT - 6)
                def _prefetch():
                    idx_start((k + 1) % 2, c + 1)
            else:
                idx_start((k + 1) % 2, c + 1)

    scatter_wait(0)
    gather_wait(2)
    scale(2, 1)
    dst_copy(2, 1)
    scatter_start(2)
    scatter_wait(1)
    scatter_wait(2)

    plsc.subcore_barrier()
    pltpu.sync_copy(acc_sh.at[pl.ds(zbase, ROWS_A)],
                    out_hbm.at[cid, pl.ds(zbase, ROWS_A)])

    @pl.when(sid == NS - 1)
    def _out_tail():
        pltpu.sync_copy(
            acc_sh.at[pl.ds(N_NODES - ROWS_TAIL, ROWS_TAIL)],
            out_hbm.at[cid, pl.ds(N_NODES - ROWS_TAIL, ROWS_TAIL)])


_R = N_NODES * EMB // 128
_BLK = 1024


def _combine_mid_body(p_ref, t_ref, ego_ref, tot_ref):
    s = p_ref[0] + p_ref[1]
    ego_ref[...] = s
    tot_ref[...] = t_ref[...] + s


def _combine_last_body(p_ref, t_ref, mean_ref):
    mean_ref[...] = (t_ref[...] + p_ref[0] + p_ref[1]) * (1.0 / (N_LAYERS + 1))


_grid = (pl.cdiv(_R, _BLK),)
_p_spec = pl.BlockSpec((NC, _BLK, 128), lambda i: (0, i, 0))
_m_spec = pl.BlockSpec((_BLK, 128), lambda i: (i, 0))

_combine_mid = pl.pallas_call(
    _combine_mid_body,
    grid=_grid,
    in_specs=[_p_spec, _m_spec],
    out_specs=[_m_spec, _m_spec],
    out_shape=[jax.ShapeDtypeStruct((_R, 128), jnp.float32)] * 2,
)

_combine_last = pl.pallas_call(
    _combine_last_body,
    grid=_grid,
    in_specs=[_p_spec, _m_spec],
    out_specs=_m_spec,
    out_shape=jax.ShapeDtypeStruct((_R, 128), jnp.float32),
)


def kernel(edge_index, edge_weight, user_emb, item_emb):
    ei = edge_index.astype(jnp.int32)
    pad = jnp.arange(N_PAD, dtype=jnp.int32) % N_NODES
    src = jnp.concatenate([ei[1], pad]).reshape(NCHUNK_P, NSTREAM, 128)
    dst = jnp.concatenate([ei[0], pad]).reshape(NCHUNK_P, NSTREAM, 128)
    w = jnp.concatenate(
        [edge_weight, jnp.zeros((N_PAD,), jnp.float32)]
    ).reshape(NCHUNK_P, CH // 16, 16)

    ego = jnp.concatenate([user_emb, item_emb], axis=0)
    tot = ego.reshape(_R, 128)
    for layer in range(N_LAYERS):
        partials = _spmm(src, dst, w, ego)
        p = partials.reshape(NC, _R, 128)
        if layer < N_LAYERS - 1:
            ego2, tot = _combine_mid(p, tot)
            ego = ego2.reshape(N_NODES, EMB)
        else:
            mean = _combine_last(p, tot).reshape(N_NODES, EMB)
    return (mean[:N_USERS], mean[N_USERS:])
